# baseline (speedup 1.0000x reference)
import functools

import jax
import jax.numpy as jnp
from jax import lax
from jax.experimental import pallas as pl
from jax.experimental.pallas import tpu as pltpu
from jax.experimental.pallas import tpu_sc as plsc

LW = 52
LH = 52
NB = 64
NT = 50
NCLS = 80
NCH = 85
CELLS = LW * LH
NTOT = NB * 3 * CELLS
A0W, A0H = 10.0 * LW / 416.0, 13.0 * LH / 416.0
A1W, A1H = 16.0 * LW / 416.0, 30.0 * LH / 416.0
A2W, A2H = 33.0 * LW / 416.0, 23.0 * LH / 416.0
IGNORE = 0.7

NLANE = NCH + 3
FLAT = NB * NT * NLANE
ROWS_PER_TILE = 72
NROWS = ROWS_PER_TILE * 32


def _veltkamp_floor_frac(v, scale):
    c = v * 4097.0
    hi = c - (c - v)
    lo = v - hi
    a = hi * scale
    b = lo * scale
    s = a + b
    n = jnp.floor(s)
    r = (a - n) + b
    n = n + (r >= 1.0).astype(jnp.float32) - (r < 0.0).astype(jnp.float32)
    frac = (a - n) + b
    return n.astype(jnp.int32), frac


def _encode_body(t0, t1, t2, t3, t4, owner_o, fx_o, fy_o, tw_o, th_o,
                 cls_o, s0_o, s1_o, s2_o, idx_o):
    v0, v1, v2, v3, v4 = t0[...], t1[...], t2[...], t3[...], t4[...]
    valid = (v0 + v1 + v2 + v3 + v4) > 0.0
    gi, fx = _veltkamp_floor_frac(v0, float(LW))
    gj, fy = _veltkamp_floor_frac(v1, float(LH))
    gw = v2 * float(LW)
    gh = v3 * float(LH)

    def iou(aw, ah):
        inter = jnp.minimum(gw, aw) * jnp.minimum(gh, ah)
        union = gw * gh + aw * ah - inter + 1e-16
        return inter / union

    i0, i1, i2 = iou(A0W, A0H), iou(A1W, A1H), iou(A2W, A2H)
    best = jnp.where(i0 >= i1, jnp.where(i0 >= i2, 0, 2),
                     jnp.where(i1 >= i2, 1, 2)).astype(jnp.int32)
    supp0 = valid & (i0 > IGNORE)
    supp1 = valid & (i1 > IGNORE)
    supp2 = valid & (i2 > IGNORE)
    aw_b = jnp.where(best == 0, A0W, jnp.where(best == 1, A1W, A2W))
    ah_b = jnp.where(best == 0, A0H, jnp.where(best == 1, A1H, A2H))

    ji = gj * LW + gi
    cell = best * CELLS + ji

    trow = lax.broadcasted_iota(jnp.int32, (NB, NT, NT), 1)
    tcol = lax.broadcasted_iota(jnp.int32, (NB, NT, NT), 2)
    same_cell = cell[:, :, None] == cell[:, None, :]
    valid_col = jnp.broadcast_to(valid[:, None, :], (NB, NT, NT))
    clobbered = jnp.any(same_cell & valid_col & (tcol > trow), axis=2)
    owner = valid & ~clobbered

    same_ji = ji[:, :, None] == ji[:, None, :]
    earlier = tcol < trow

    def first_of(supp_a):
        col = jnp.broadcast_to(supp_a[:, None, :], (NB, NT, NT))
        return supp_a & ~jnp.any(same_ji & earlier & col, axis=2)

    f0, f1, f2 = first_of(supp0), first_of(supp1), first_of(supp2)

    owner_o[...] = owner.astype(jnp.float32)
    fx_o[...] = fx
    fy_o[...] = fy
    tw_o[...] = gw / aw_b
    th_o[...] = gh / ah_b
    cls_o[...] = v4.astype(jnp.int32)
    s0_o[...] = f0.astype(jnp.float32)
    s1_o[...] = f1.astype(jnp.float32)
    s2_o[...] = f2.astype(jnp.float32)

    k = lax.broadcasted_iota(jnp.int32, (NB, NT, NLANE), 2)
    b = lax.broadcasted_iota(jnp.int32, (NB, NT, NLANE), 0)
    best3 = jnp.broadcast_to(best[:, :, None], (NB, NT, NLANE))
    ji3 = jnp.broadcast_to(ji[:, :, None], (NB, NT, NLANE))
    ch = jnp.where(k < NCH, NCH * best3 + k, NCH * (k - NCH) + 4)
    idx_o[...] = (b * 255 + ch) * CELLS + ji3


def _encode(target):
    f2 = jax.ShapeDtypeStruct((NB, NT), jnp.float32)
    i2 = jax.ShapeDtypeStruct((NB, NT), jnp.int32)
    outs = [f2, f2, f2, f2, f2, i2, f2, f2, f2,
            jax.ShapeDtypeStruct((NB, NT, NLANE), jnp.int32)]
    slices = [target[:, :, i] for i in range(5)]
    return pl.pallas_call(_encode_body, out_shape=outs)(*slices)


def _gather_tile(flat_hbm, idx_hbm, out_hbm, idx_v, rows_v, sem):
    wid = lax.axis_index("s") * 2 + lax.axis_index("c")
    r0 = wid * ROWS_PER_TILE
    pltpu.sync_copy(idx_hbm.at[pl.ds(r0, ROWS_PER_TILE)], idx_v)
    def chunk(g, carry):
        copies = []
        for b in range(8):
            j = g * 8 + b
            copies.append(pltpu.make_async_copy(
                flat_hbm.at[idx_v.at[j]], rows_v.at[j], sem))
        for c in copies:
            c.start()
        for c in copies:
            c.wait()
        return carry
    lax.fori_loop(0, ROWS_PER_TILE // 8, chunk, 0)
    pltpu.sync_copy(rows_v, out_hbm.at[pl.ds(r0, ROWS_PER_TILE)])


def _gather(flat_input, idx2d):
    mesh = plsc.VectorSubcoreMesh(core_axis_name="c", subcore_axis_name="s")
    kfn = functools.partial(
        pl.kernel,
        mesh=mesh,
        out_type=jax.ShapeDtypeStruct((NROWS, 128), jnp.float32),
        scratch_types=[
            pltpu.VMEM((ROWS_PER_TILE, 128), jnp.int32),
            pltpu.VMEM((ROWS_PER_TILE, 128), jnp.float32),
            pltpu.SemaphoreType.DMA,
        ],
    )(_gather_tile)
    return kfn(flat_input, idx2d)


def _final_body(g_ref, owner_ref, fx_ref, fy_ref, tw_ref, th_ref, cls_ref,
                s0_ref, s1_ref, s2_ref, confz_ref, o_ref):
    g = g_ref[...]
    of = owner_ref[...]
    k = lax.broadcasted_iota(jnp.int32, (NB, NT, NLANE), 2)

    sig = jax.nn.sigmoid(g)
    logp = jnp.maximum(jnp.log(sig), -100.0)
    log1mp = jnp.maximum(jnp.log(1.0 - sig), -100.0)
    ex = jnp.exp(g)

    def b3(x):
        return jnp.broadcast_to(x[:, :, None], (NB, NT, NLANE))

    of3 = b3(of)
    pred = jnp.where(k < 2, sig, ex)
    tgt = jnp.where(k == 0, b3(fx_ref[...]),
                    jnp.where(k == 1, b3(fy_ref[...]),
                              jnp.where(k == 2, b3(tw_ref[...]), b3(th_ref[...]))))
    mse = of3 * (pred - tgt) * (pred - tgt)
    obj = of3 * (-logp)
    supp = jnp.where(k == NCH, b3(s0_ref[...]),
                     jnp.where(k == NCH + 1, b3(s1_ref[...]), b3(s2_ref[...])))
    noobj_corr = supp * (-log1mp)
    wk = jnp.where(k < 4, 5.0 / NTOT,
                   jnp.where(k == 4, 1.0 / NTOT, -1.0 / NTOT))
    main = jnp.where(k < 4, mse, jnp.where(k == 4, obj, noobj_corr)) * wk
    sum_main = jnp.sum(jnp.where((k < 5) | (k >= NCH), main, 0.0))

    tcls = (k - 5) == b3(cls_ref[...])
    bce = -jnp.where(tcls, logp, log1mp)
    sum_cls = jnp.sum(jnp.where((k >= 5) & (k < NCH), of3 * bce, 0.0))

    npos = jnp.sum(of)

    pz = jax.nn.sigmoid(confz_ref[...])
    sall = jnp.sum(-jnp.maximum(jnp.log(1.0 - pz), -100.0))

    loss = (sum_main + sall / NTOT + sum_cls / (npos * NCLS)) * NB
    o_ref[0, 0] = loss


def _final(g, owner, fx, fy, tw, th, cls, s0, s1, s2, confz):
    return pl.pallas_call(
        _final_body,
        in_specs=[pl.BlockSpec(memory_space=pltpu.VMEM)] * 11,
        out_specs=pl.BlockSpec(memory_space=pltpu.SMEM),
        out_shape=jax.ShapeDtypeStruct((1, 1), jnp.float32),
    )(g, owner, fx, fy, tw, th, cls, s0, s1, s2, confz)


def kernel(input, target):
    owner, fx, fy, tw, th, cls, s0, s1, s2, idx = _encode(target)
    pad = jnp.zeros((NROWS * 128 - FLAT,), jnp.int32)
    idx2d = jnp.concatenate([idx.reshape(-1), pad]).reshape(NROWS, 128)
    g = _gather(input.reshape(-1), idx2d)
    g = g.reshape(-1)[:FLAT].reshape(NB, NT, NLANE)
    confz = input[:, 4::NCH, :, :]
    out = _final(g, owner, fx, fy, tw, th, cls, s0, s1, s2, confz)
    return out[0, 0]


# speedup vs baseline: 1.7046x; 1.7046x over previous
import functools

import jax
import jax.numpy as jnp
from jax import lax
from jax.experimental import pallas as pl
from jax.experimental.pallas import tpu as pltpu
from jax.experimental.pallas import tpu_sc as plsc

LW = 52
LH = 52
NB = 64
NT = 50
NTP = 64
NCLS = 80
NCH = 85
NTOT = NB * 3 * LH * LW
A0W, A0H = 10.0 * LW / 416.0, 13.0 * LH / 416.0
A1W, A1H = 16.0 * LW / 416.0, 30.0 * LH / 416.0
A2W, A2H = 33.0 * LW / 416.0, 23.0 * LH / 416.0
IGNORE = 0.7

NG = 7
PLROWS = 56
PL_ELEMS = 192 * PLROWS * 128
GRID_ELEMS = NG * PL_ELEMS
ZCH = 3 * PLROWS * 128


def _veltkamp_floor_frac(v, scale):
    c = v * 4097.0
    hi = c - (c - v)
    lo = v - hi
    a = hi * scale
    b = lo * scale
    s = a + b
    n = jnp.floor(s)
    r = (a - n) + b
    n = n + (r >= 1.0).astype(jnp.float32) - (r < 0.0).astype(jnp.float32)
    frac = (a - n) + b
    return n.astype(jnp.int32), frac


def _encode_body(t0, t1, t2, t3, t4, fx_o, fy_o, tw_o, th_o, cls_o,
                 oidx_o, s0_o, s1_o, s2_o):
    v0, v1, v2, v3, v4 = t0[...], t1[...], t2[...], t3[...], t4[...]
    valid = (v0 + v1 + v2 + v3 + v4) > 0.0
    gi, fx = _veltkamp_floor_frac(v0, float(LW))
    gj, fy = _veltkamp_floor_frac(v1, float(LH))
    gw = v2 * float(LW)
    gh = v3 * float(LH)

    def iou(aw, ah):
        inter = jnp.minimum(gw, aw) * jnp.minimum(gh, ah)
        union = gw * gh + aw * ah - inter + 1e-16
        return inter / union

    i0, i1, i2 = iou(A0W, A0H), iou(A1W, A1H), iou(A2W, A2H)
    best = jnp.where(i0 >= i1, jnp.where(i0 >= i2, 0, 2),
                     jnp.where(i1 >= i2, 1, 2)).astype(jnp.int32)
    supp0 = valid & (i0 > IGNORE)
    supp1 = valid & (i1 > IGNORE)
    supp2 = valid & (i2 > IGNORE)
    aw_b = jnp.where(best == 0, A0W, jnp.where(best == 1, A1W, A2W))
    ah_b = jnp.where(best == 0, A0H, jnp.where(best == 1, A1H, A2H))

    cell = best * (LH * LW) + gj * LW + gi

    trow = lax.broadcasted_iota(jnp.int32, (NB, NTP, NTP), 1)
    tcol = lax.broadcasted_iota(jnp.int32, (NB, NTP, NTP), 2)
    same_cell = cell[:, :, None] == cell[:, None, :]
    valid_col = jnp.broadcast_to(valid[:, None, :], (NB, NTP, NTP))
    clobbered = jnp.any(same_cell & valid_col & (tcol > trow), axis=2)
    owner = valid & ~clobbered

    fx_o[...] = fx
    fy_o[...] = fy
    tw_o[...] = gw / aw_b
    th_o[...] = gh / ah_b
    cls_o[...] = v4.astype(jnp.int32).astype(jnp.float32)

    b = lax.broadcasted_iota(jnp.int32, (NB, NTP), 0)
    dummy = b * ZCH + 127
    base = ((b * 3 + best) * PLROWS + gj) * 128 + gi
    oidx_o[...] = jnp.where(owner, base, dummy)
    ji = gj * 128 + gi
    s0_o[...] = jnp.where(supp0, (b * 3 + 0) * PLROWS * 128 + ji, dummy)
    s1_o[...] = jnp.where(supp1, (b * 3 + 1) * PLROWS * 128 + ji, dummy)
    s2_o[...] = jnp.where(supp2, (b * 3 + 2) * PLROWS * 128 + ji, dummy)


def _encode(target):
    f2 = jax.ShapeDtypeStruct((NB, NTP), jnp.float32)
    i2 = jax.ShapeDtypeStruct((NB, NTP), jnp.int32)
    outs = [f2, f2, f2, f2, f2, i2, i2, i2, i2]
    tpad = jnp.pad(target, ((0, 0), (0, NTP - NT), (0, 0)))
    slices = [tpad[:, :, i] for i in range(5)]
    return pl.pallas_call(_encode_body, out_shape=outs)(*slices)


def _scatter_tile(fx_h, fy_h, tw_h, th_h, cls_h, oidx_h, s0_h, s1_h, s2_h,
                  out_h, zbuf, ones_v, vals, idxs, semi, semz, sems):
    wid = lax.axis_index("s") * 2 + lax.axis_index("c")
    r0 = wid * 2

    loads = []
    for i, src in enumerate((fx_h, fy_h, tw_h, th_h, cls_h)):
        loads.append(pltpu.make_async_copy(src.at[pl.ds(r0, 2)], vals[i], semi))
    base_idx = [oidx_h, s0_h, s1_h, s2_h]
    for i, src in enumerate(base_idx):
        loads.append(pltpu.make_async_copy(src.at[pl.ds(r0, 2)], idxs[i], semi))
    for c in loads:
        c.start()

    def zfill(i, carry):
        zbuf[pl.ds(i * 16, 16)] = jnp.zeros((16,), jnp.float32)
        return carry
    lax.fori_loop(0, ZCH // 16, zfill, 0)
    zcopies = []
    for g in range(NG):
        for bb in range(2):
            off = g * PL_ELEMS + (r0 + bb) * ZCH
            zcopies.append(pltpu.make_async_copy(
                zbuf, out_h.at[pl.ds(off, ZCH)], semz))
    for c in zcopies:
        c.start()

    for r in range(2):
        for c in range(NTP // 16):
            ones_v[r, pl.ds(c * 16, 16)] = jnp.full((16,), 1.0, jnp.float32)
    for c in loads:
        c.wait()

    def shift(dst, src, off):
        for r in range(2):
            for c in range(4):
                dst[r, pl.ds(c * 16, 16)] = src[r, pl.ds(c * 16, 16)] + off
    for g in range(1, 6):
        shift(idxs[3 + g], idxs[0], g * PL_ELEMS)
    shift(idxs[9], idxs[1], 6 * PL_ELEMS)
    shift(idxs[10], idxs[2], 6 * PL_ELEMS)
    shift(idxs[11], idxs[3], 6 * PL_ELEMS)

    for c in zcopies:
        c.wait()

    scs = []
    for r in range(2):
        scs.append(pltpu.make_async_copy(
            ones_v.at[r], out_h.at[idxs[0].at[r]], sems))
        for g in range(1, 6):
            scs.append(pltpu.make_async_copy(
                vals[g - 1].at[r], out_h.at[idxs[3 + g].at[r]], sems))
        for i in range(3):
            scs.append(pltpu.make_async_copy(
                ones_v.at[r], out_h.at[idxs[9 + i].at[r]], sems))
    for c in scs:
        c.start()
    for c in scs:
        c.wait()


def _scatter(fx, fy, tw, th, cls, oidx, s0, s1, s2):
    mesh = plsc.VectorSubcoreMesh(core_axis_name="c", subcore_axis_name="s")
    kfn = functools.partial(
        pl.kernel,
        mesh=mesh,
        out_type=jax.ShapeDtypeStruct((GRID_ELEMS,), jnp.float32),
        scratch_types=[
            pltpu.VMEM((ZCH,), jnp.float32),
            pltpu.VMEM((2, NTP), jnp.float32),
            [pltpu.VMEM((2, NTP), jnp.float32) for _ in range(5)],
            [pltpu.VMEM((2, NTP), jnp.int32) for _ in range(12)],
            pltpu.SemaphoreType.DMA,
            pltpu.SemaphoreType.DMA,
            pltpu.SemaphoreType.DMA,
        ],
    )(_scatter_tile)
    return kfn(fx, fy, tw, th, cls, oidx, s0, s1, s2)


def _dense_body(x_ref, m_ref, tx_ref, ty_ref, tw_ref, th_ref, cls_ref,
                sp_ref, o_ref):
    b = pl.program_id(0)
    a = pl.program_id(1)

    @pl.when((b == 0) & (a == 0))
    def _():
        for i in range(8):
            o_ref[0, i] = 0.0

    z = x_ref[0]
    m = m_ref[:LH, :LW]
    txg = tx_ref[:LH, :LW]
    tyg = ty_ref[:LH, :LW]
    twg = tw_ref[:LH, :LW]
    thg = th_ref[:LH, :LW]
    clsg = cls_ref[:LH, :LW].astype(jnp.int32)
    sup = sp_ref[:LH, :LW]

    sig = jax.nn.sigmoid
    x = sig(z[0])
    y = sig(z[1])
    wq = jnp.exp(z[2])
    hq = jnp.exp(z[3])
    p4 = sig(z[4])

    mse = ((x * m - txg * m) ** 2 + (y * m - tyg * m) ** 2
           + (wq * m - twg * m) ** 2 + (hq * m - thg * m) ** 2)

    pm = p4 * m
    obj = -(m * jnp.maximum(jnp.log(pm), -100.0)
            + (1.0 - m) * jnp.maximum(jnp.log(1.0 - pm), -100.0))

    pn = p4 * (1.0 - sup)
    noobj = -jnp.maximum(jnp.log(1.0 - pn), -100.0)

    zc = z[5:]
    pc = sig(zc)
    logp = jnp.maximum(jnp.log(pc), -100.0)
    log1mp = jnp.maximum(jnp.log(1.0 - pc), -100.0)
    k = lax.broadcasted_iota(jnp.int32, (NCLS, LH, LW), 0)
    tcls = k == clsg[None, :, :]
    per = -jnp.where(tcls, logp, log1mp)
    clsum = jnp.sum(per * m[None, :, :])

    o_ref[0, 0] += jnp.sum(mse)
    o_ref[0, 1] += jnp.sum(obj)
    o_ref[0, 2] += jnp.sum(noobj)
    o_ref[0, 3] += clsum
    o_ref[0, 4] += jnp.sum(m)

    @pl.when((b == NB - 1) & (a == 2))
    def _():
        box = 5.0 * o_ref[0, 0] / NTOT
        objl = o_ref[0, 1] / NTOT
        noobjl = o_ref[0, 2] / NTOT
        clsl = o_ref[0, 3] / (o_ref[0, 4] * NCLS)
        o_ref[0, 5] = (box + objl + noobjl + clsl) * NB


def _dense(input, grids2d):
    def gspec(g):
        return pl.BlockSpec((PLROWS, 128),
                            lambda b, a, g=g: (g * 192 + b * 3 + a, 0))
    return pl.pallas_call(
        _dense_body,
        grid=(NB, 3),
        in_specs=[pl.BlockSpec((1, NCH, LH, LW), lambda b, a: (b, a, 0, 0))]
        + [gspec(g) for g in range(NG)],
        out_specs=pl.BlockSpec((1, 8), lambda b, a: (0, 0),
                               memory_space=pltpu.SMEM),
        out_shape=jax.ShapeDtypeStruct((1, 8), jnp.float32),
    )(input, *([grids2d] * NG))


def kernel(input, target):
    fx, fy, tw, th, cls, oidx, s0, s1, s2 = _encode(target)
    grids = _scatter(fx, fy, tw, th, cls, oidx, s0, s1, s2)
    grids2d = grids.reshape(NG * 192 * PLROWS, 128)
    out = _dense(input, grids2d)
    return out[0, 5]
